# baseline (speedup 1.0000x reference)
import functools

import jax
import jax.numpy as jnp
from jax import lax
from jax.experimental import pallas as pl
from jax.experimental.pallas import tpu as pltpu
from jax.experimental.pallas import tpu_sc as plsc

H = 1024
FF = 2048
E = 8
FFE = FF // E
TT = 256

NC = 2
NS = 16
NW = NC * NS
L = 16


def _router_body(x_ref, gw_ref, pe_ref, pe2_ref, pw_ref, pw2_ref):
    x = x_ref[...]
    logits = jnp.dot(x, gw_ref[...], preferred_element_type=jnp.float32)
    eidx = lax.broadcasted_iota(jnp.int32, logits.shape, 1)
    v1 = jnp.max(logits, axis=-1, keepdims=True)
    i1 = jnp.min(jnp.where(logits == v1, eidx, E), axis=-1, keepdims=True)
    lm = jnp.where(eidx == i1, -jnp.inf, logits)
    v2 = jnp.max(lm, axis=-1, keepdims=True)
    i2 = jnp.min(jnp.where(lm == v2, eidx, E), axis=-1, keepdims=True)
    w1 = 1.0 / (1.0 + jnp.exp(v2 - v1))
    pe_ref[...] = i1
    pe2_ref[...] = i2
    pw_ref[...] = w1
    pw2_ref[...] = 1.0 - w1


def _router(x_flat, gate_weight):
    T = x_flat.shape[0]
    return pl.pallas_call(
        _router_body,
        out_shape=[
            jax.ShapeDtypeStruct((T, 1), jnp.int32),
            jax.ShapeDtypeStruct((T, 1), jnp.int32),
            jax.ShapeDtypeStruct((T, 1), jnp.float32),
            jax.ShapeDtypeStruct((T, 1), jnp.float32),
        ],
    )(x_flat, gate_weight)


def _dispatch_body(P, PAD, pe_hbm, pos_hbm, src_hbm, te_hbm,
                   pe_v, rank_v, dst_v, st_v, init_v, allh_v, te_v,
                   hist_r, base_r, sh_hists, sh_src):
    cid = lax.axis_index("c")
    sid = lax.axis_index("s")
    ppw = P // NS
    spw = PAD // NS
    lane = lax.iota(jnp.int32, L)

    @pl.when(cid == 0)
    def _():
        base_pair = sid * ppw
        pltpu.sync_copy(pe_hbm.at[pl.ds(base_pair, ppw)], pe_v)
        hist_r[...] = jnp.zeros((L,), jnp.int32)

        @pl.loop(0, ppw // L)
        def _(j):
            v = pe_v[pl.ds(j * L, L)]
            base = plsc.load_gather(hist_r, [v])
            rank_in = jnp.zeros((L,), jnp.int32)
            cntv = jnp.zeros((L,), jnp.int32)
            for e in range(E):
                m = v == e
                c = plsc.cumsum(m.astype(jnp.int32))
                rank_in = jnp.where(m, c - 1, rank_in)
                tote = jnp.max(c)
                cntv = cntv + jnp.where(lane == e, 1, 0) * tote
            rank_v[pl.ds(j * L, L)] = base + rank_in
            hist_r[...] = hist_r[...] + cntv

        pltpu.sync_copy(hist_r, sh_hists.at[sid])
        plsc.subcore_barrier()
        pltpu.sync_copy(sh_hists, allh_v)
        tot = jnp.zeros((L,), jnp.int32)
        prevc = jnp.zeros((L,), jnp.int32)
        for w2 in range(NS):
            hw = allh_v[w2]
            tot = tot + hw
            prevc = prevc + hw * (w2 < sid).astype(jnp.int32)
        pcnt = (tot + (TT - 1)) & (-TT)
        segend = plsc.cumsum(pcnt)
        segstart = segend - pcnt
        base_r[...] = segstart + prevc

        @pl.loop(0, ppw // L)
        def _(j):
            v = pe_v[pl.ds(j * L, L)]
            r = rank_v[pl.ds(j * L, L)]
            bgl = plsc.load_gather(base_r, [v])
            dst_v[pl.ds(j * L, L)] = bgl + r
            pvec = base_pair + j * L + lane
            st_v[pl.ds(j * L, L)] = pvec & (P // 2 - 1)

        pltpu.sync_copy(dst_v, pos_hbm.at[pl.ds(base_pair, ppw)])

        base_slot = sid * spw

        @pl.loop(0, spw // L)
        def _(j):
            init_v[pl.ds(j * L, L)] = (base_slot + j * L + lane) & (P // 2 - 1)

        pltpu.sync_copy(init_v, sh_src.at[pl.ds(base_slot, spw)])
        plsc.subcore_barrier()

        @pl.loop(0, ppw // L)
        def _(j):
            idxv = dst_v[pl.ds(j * L, L)]
            pltpu.sync_copy(st_v.at[pl.ds(j * L, L)], sh_src.at[idxv])

        plsc.subcore_barrier()
        pltpu.sync_copy(sh_src.at[pl.ds(base_slot, spw)],
                        src_hbm.at[pl.ds(base_slot, spw)])

        @pl.when(sid == 0)
        def _():
            ntiles = PAD // TT
            nvec = (ntiles + L - 1) // L
            for k in range(nvec):
                tstart = (k * L + lane) * TT
                cnt = jnp.zeros((L,), jnp.int32)
                for e in range(E):
                    se = jnp.max(jnp.where(lane == e, segend, 0))
                    cnt = cnt + (se <= tstart).astype(jnp.int32)
                te_v[pl.ds(k * L, L)] = jnp.minimum(cnt, E - 1)
            pltpu.sync_copy(te_v, te_hbm)


def _dispatch(pe_flat):
    P = pe_flat.shape[0]
    PAD = (P // TT + E) * TT
    ntiles = PAD // TT
    te_n = ((ntiles + L - 1) // L) * L
    mesh = plsc.VectorSubcoreMesh(core_axis_name="c", subcore_axis_name="s", num_cores=NC, num_subcores=NS)
    f = pl.kernel(
        functools.partial(_dispatch_body, P, PAD),
        compiler_params=pltpu.CompilerParams(needs_layout_passes=False),
        out_type=[
            jax.ShapeDtypeStruct((P,), jnp.int32),
            jax.ShapeDtypeStruct((PAD,), jnp.int32),
            jax.ShapeDtypeStruct((te_n,), jnp.int32),
        ],
        mesh=mesh,
        scratch_types=[
            pltpu.VMEM((P // NS,), jnp.int32),
            pltpu.VMEM((P // NS,), jnp.int32),
            pltpu.VMEM((P // NS,), jnp.int32),
            pltpu.VMEM((P // NS,), jnp.int32),
            pltpu.VMEM((PAD // NS,), jnp.int32),
            pltpu.VMEM((NS, L), jnp.int32),
            pltpu.VMEM((te_n,), jnp.int32),
            pltpu.VMEM((L,), jnp.int32),
            pltpu.VMEM((L,), jnp.int32),
            pltpu.VMEM_SHARED((NS, L), jnp.int32),
            pltpu.VMEM_SHARED((PAD,), jnp.int32),
        ],
    )
    return f(pe_flat)


def _gather_body(PAD, src_hbm, x_hbm, xs_hbm, src_v, xrow0, xrow1, xrow2, xrow3, sem):
    cid = lax.axis_index("c")
    sid = lax.axis_index("s")
    wid = sid * NC + cid
    spw = PAD // NW
    base = wid * spw
    pltpu.sync_copy(src_hbm.at[pl.ds(base, spw)], src_v)
    bufs = (xrow0, xrow1, xrow2, xrow3)
    nb = len(bufs)

    @pl.loop(0, spw // (L * nb))
    def _(g):
        j0 = g * nb
        started = []
        for b in range(nb):
            idxv = src_v[pl.ds((j0 + b) * L, L)]
            started.append(pltpu.async_copy(x_hbm.at[idxv], bufs[b], sem))
        for b in range(nb):
            started[b].wait()
            pltpu.sync_copy(bufs[b], xs_hbm.at[pl.ds(base + (j0 + b) * L, L)])


def _gather(src, x_flat):
    PAD = src.shape[0]
    mesh = plsc.VectorSubcoreMesh(core_axis_name="c", subcore_axis_name="s", num_cores=NC, num_subcores=NS)
    f = pl.kernel(
        functools.partial(_gather_body, PAD),
        compiler_params=pltpu.CompilerParams(needs_layout_passes=False),
        out_type=jax.ShapeDtypeStruct((PAD, H), jnp.float32),
        mesh=mesh,
        scratch_types=[
            pltpu.VMEM((PAD // NW,), jnp.int32),
            pltpu.VMEM((L, H), jnp.float32),
            pltpu.VMEM((L, H), jnp.float32),
            pltpu.VMEM((L, H), jnp.float32),
            pltpu.VMEM((L, H), jnp.float32),
            pltpu.SemaphoreType.DMA,
        ],
    )
    return f(src, x_flat)


def _gemm_body(te_ref, xs_ref, wg_ref, wu_ref, wd_ref, y_ref):
    xw = xs_ref[...].astype(jnp.bfloat16)
    hg = jnp.dot(xw, wg_ref[0], preferred_element_type=jnp.float32)
    hu = jnp.dot(xw, wu_ref[0], preferred_element_type=jnp.float32)
    h1 = hg * jax.nn.sigmoid(hg) * hu
    y_ref[...] = jnp.dot(
        h1.astype(jnp.bfloat16), wd_ref[0], preferred_element_type=jnp.float32
    )


def _gemm(te, xs, W_gate, W_up, W_down):
    PAD = xs.shape[0]
    ntiles = PAD // TT
    grid_spec = pltpu.PrefetchScalarGridSpec(
        num_scalar_prefetch=1,
        grid=(ntiles,),
        in_specs=[
            pl.BlockSpec((TT, H), lambda i, te_ref: (i, 0)),
            pl.BlockSpec((1, H, FFE), lambda i, te_ref: (te_ref[i], 0, 0)),
            pl.BlockSpec((1, H, FFE), lambda i, te_ref: (te_ref[i], 0, 0)),
            pl.BlockSpec((1, FFE, H), lambda i, te_ref: (te_ref[i], 0, 0)),
        ],
        out_specs=pl.BlockSpec((TT, H), lambda i, te_ref: (i, 0)),
    )
    return pl.pallas_call(
        _gemm_body,
        grid_spec=grid_spec,
        out_shape=jax.ShapeDtypeStruct((PAD, H), jnp.float32),
    )(te, xs, W_gate, W_up, W_down)


def _combine_body(T, pos_hbm, pw_hbm, y_hbm, out_hbm,
                  pos1_v, pos2_v, pw1_v, pw2_v, y1_v, y2_v, out_v, sem1, sem2):
    cid = lax.axis_index("c")
    sid = lax.axis_index("s")
    wid = sid * NC + cid
    tpw = T // NW
    tb = wid * tpw
    pltpu.sync_copy(pos_hbm.at[pl.ds(tb, tpw)], pos1_v)
    pltpu.sync_copy(pos_hbm.at[pl.ds(T + tb, tpw)], pos2_v)
    pltpu.sync_copy(pw_hbm.at[pl.ds(tb, tpw)], pw1_v.at[pl.ds(0, tpw)])
    pltpu.sync_copy(pw_hbm.at[pl.ds(T + tb, tpw)], pw2_v.at[pl.ds(0, tpw)])

    @pl.loop(0, tpw // L)
    def _(jc):
        i1v = pos1_v[pl.ds(jc * L, L)]
        i2v = pos2_v[pl.ds(jc * L, L)]
        d1 = pltpu.async_copy(y_hbm.at[i1v], y1_v, sem1)
        d2 = pltpu.async_copy(y_hbm.at[i2v], y2_v, sem2)
        d1.wait()
        d2.wait()
        for r in range(L):
            w1 = pw1_v[pl.ds(jc * L + r, L)][0]
            w2 = pw2_v[pl.ds(jc * L + r, L)][0]

            @pl.loop(0, H // L)
            def _(c):
                out_v[r, pl.ds(c * L, L)] = (
                    y1_v[r, pl.ds(c * L, L)] * w1 + y2_v[r, pl.ds(c * L, L)] * w2
                )
        pltpu.sync_copy(out_v, out_hbm.at[pl.ds(tb + jc * L, L)])


def _combine(pos, pw_flat, y, T):
    mesh = plsc.VectorSubcoreMesh(core_axis_name="c", subcore_axis_name="s", num_cores=NC, num_subcores=NS)
    f = pl.kernel(
        functools.partial(_combine_body, T),
        compiler_params=pltpu.CompilerParams(needs_layout_passes=False),
        out_type=jax.ShapeDtypeStruct((T, H), jnp.float32),
        mesh=mesh,
        scratch_types=[
            pltpu.VMEM((T // NW,), jnp.int32),
            pltpu.VMEM((T // NW,), jnp.int32),
            pltpu.VMEM((T // NW + L,), jnp.float32),
            pltpu.VMEM((T // NW + L,), jnp.float32),
            pltpu.VMEM((L, H), jnp.float32),
            pltpu.VMEM((L, H), jnp.float32),
            pltpu.VMEM((L, H), jnp.float32),
            pltpu.SemaphoreType.DMA,
            pltpu.SemaphoreType.DMA,
        ],
    )
    return f(pos, pw_flat, y)


@jax.jit
def kernel(x, gate_weight, W_gate, W_up, W_down):
    b, s, h = x.shape
    T = b * s
    x_flat = x.reshape(T, h)
    pe1, pe2, pw1, pw2 = _router(x_flat, gate_weight)
    pe_flat = jnp.concatenate([pe1.reshape(-1), pe2.reshape(-1)])
    pw_flat = jnp.concatenate([pw1.reshape(-1), pw2.reshape(-1)])
    pos, src, te = _dispatch(pe_flat)
    xs = _gather(src, x_flat)
    y = _gemm(te, xs, W_gate.astype(jnp.bfloat16), W_up.astype(jnp.bfloat16),
              W_down.astype(jnp.bfloat16))
    out = _combine(pos, pw_flat, y, T)
    return out.reshape(b, s, h)


# speedup vs baseline: 1.0265x; 1.0265x over previous
import functools

import jax
import jax.numpy as jnp
from jax import lax
from jax.experimental import pallas as pl
from jax.experimental.pallas import tpu as pltpu
from jax.experimental.pallas import tpu_sc as plsc

H = 1024
FF = 2048
E = 8
FFE = FF // E
TT = 256

NC = 2
NS = 16
NW = NC * NS
L = 16


def _router_body(x_ref, gw_ref, pe_ref, pe2_ref, pw_ref, pw2_ref):
    x = x_ref[...]
    logits = jnp.dot(x, gw_ref[...], preferred_element_type=jnp.float32)
    eidx = lax.broadcasted_iota(jnp.int32, logits.shape, 1)
    v1 = jnp.max(logits, axis=-1, keepdims=True)
    i1 = jnp.min(jnp.where(logits == v1, eidx, E), axis=-1, keepdims=True)
    lm = jnp.where(eidx == i1, -jnp.inf, logits)
    v2 = jnp.max(lm, axis=-1, keepdims=True)
    i2 = jnp.min(jnp.where(lm == v2, eidx, E), axis=-1, keepdims=True)
    w1 = 1.0 / (1.0 + jnp.exp(v2 - v1))
    pe_ref[...] = i1
    pe2_ref[...] = i2
    pw_ref[...] = w1
    pw2_ref[...] = 1.0 - w1


def _router(x_flat, gate_weight):
    T = x_flat.shape[0]
    return pl.pallas_call(
        _router_body,
        out_shape=[
            jax.ShapeDtypeStruct((T, 1), jnp.int32),
            jax.ShapeDtypeStruct((T, 1), jnp.int32),
            jax.ShapeDtypeStruct((T, 1), jnp.float32),
            jax.ShapeDtypeStruct((T, 1), jnp.float32),
        ],
    )(x_flat, gate_weight)


def _dispatch_body(P, PAD, pe_hbm, pos_hbm, src_hbm, te_hbm,
                   pe_v, rank_v, dst_v, st_v, init_v, allh_v, te_v,
                   hist_r, base_r, sh_hists, sh_src):
    cid = lax.axis_index("c")
    sid = lax.axis_index("s")
    ppw = P // NS
    spw = PAD // NS
    lane = lax.iota(jnp.int32, L)

    @pl.when(cid == 0)
    def _():
        base_pair = sid * ppw
        pltpu.sync_copy(pe_hbm.at[pl.ds(base_pair, ppw)], pe_v)
        hist_r[...] = jnp.zeros((L,), jnp.int32)

        @pl.loop(0, ppw // L)
        def _(j):
            v = pe_v[pl.ds(j * L, L)]
            base = plsc.load_gather(hist_r, [v])
            rank_in = jnp.zeros((L,), jnp.int32)
            cntv = jnp.zeros((L,), jnp.int32)
            for e in range(E):
                m = v == e
                c = plsc.cumsum(m.astype(jnp.int32))
                rank_in = jnp.where(m, c - 1, rank_in)
                tote = jnp.max(c)
                cntv = cntv + jnp.where(lane == e, 1, 0) * tote
            rank_v[pl.ds(j * L, L)] = base + rank_in
            hist_r[...] = hist_r[...] + cntv

        pltpu.sync_copy(hist_r, sh_hists.at[sid])
        plsc.subcore_barrier()
        pltpu.sync_copy(sh_hists, allh_v)
        tot = jnp.zeros((L,), jnp.int32)
        prevc = jnp.zeros((L,), jnp.int32)
        for w2 in range(NS):
            hw = allh_v[w2]
            tot = tot + hw
            prevc = prevc + hw * (w2 < sid).astype(jnp.int32)
        pcnt = (tot + (TT - 1)) & (-TT)
        segend = plsc.cumsum(pcnt)
        segstart = segend - pcnt
        base_r[...] = segstart + prevc

        @pl.loop(0, ppw // L)
        def _(j):
            v = pe_v[pl.ds(j * L, L)]
            r = rank_v[pl.ds(j * L, L)]
            bgl = plsc.load_gather(base_r, [v])
            dst_v[pl.ds(j * L, L)] = bgl + r
            pvec = base_pair + j * L + lane
            st_v[pl.ds(j * L, L)] = pvec & (P // 2 - 1)

        pltpu.sync_copy(dst_v, pos_hbm.at[pl.ds(base_pair, ppw)])

        base_slot = sid * spw

        @pl.loop(0, spw // L)
        def _(j):
            init_v[pl.ds(j * L, L)] = (base_slot + j * L + lane) & (P // 2 - 1)

        pltpu.sync_copy(init_v, sh_src.at[pl.ds(base_slot, spw)])
        plsc.subcore_barrier()

        @pl.loop(0, ppw // L)
        def _(j):
            idxv = dst_v[pl.ds(j * L, L)]
            pltpu.sync_copy(st_v.at[pl.ds(j * L, L)], sh_src.at[idxv])

        plsc.subcore_barrier()
        pltpu.sync_copy(sh_src.at[pl.ds(base_slot, spw)],
                        src_hbm.at[pl.ds(base_slot, spw)])

        @pl.when(sid == 0)
        def _():
            ntiles = PAD // TT
            nvec = (ntiles + L - 1) // L
            for k in range(nvec):
                tstart = (k * L + lane) * TT
                cnt = jnp.zeros((L,), jnp.int32)
                for e in range(E):
                    se = jnp.max(jnp.where(lane == e, segend, 0))
                    cnt = cnt + (se <= tstart).astype(jnp.int32)
                te_v[pl.ds(k * L, L)] = jnp.minimum(cnt, E - 1)
            pltpu.sync_copy(te_v, te_hbm)


def _dispatch(pe_flat):
    P = pe_flat.shape[0]
    PAD = (P // TT + E) * TT
    ntiles = PAD // TT
    te_n = ((ntiles + L - 1) // L) * L
    mesh = plsc.VectorSubcoreMesh(core_axis_name="c", subcore_axis_name="s", num_cores=NC, num_subcores=NS)
    f = pl.kernel(
        functools.partial(_dispatch_body, P, PAD),
        compiler_params=pltpu.CompilerParams(needs_layout_passes=False),
        out_type=[
            jax.ShapeDtypeStruct((P,), jnp.int32),
            jax.ShapeDtypeStruct((PAD,), jnp.int32),
            jax.ShapeDtypeStruct((te_n,), jnp.int32),
        ],
        mesh=mesh,
        scratch_types=[
            pltpu.VMEM((P // NS,), jnp.int32),
            pltpu.VMEM((P // NS,), jnp.int32),
            pltpu.VMEM((P // NS,), jnp.int32),
            pltpu.VMEM((P // NS,), jnp.int32),
            pltpu.VMEM((PAD // NS,), jnp.int32),
            pltpu.VMEM((NS, L), jnp.int32),
            pltpu.VMEM((te_n,), jnp.int32),
            pltpu.VMEM((L,), jnp.int32),
            pltpu.VMEM((L,), jnp.int32),
            pltpu.VMEM_SHARED((NS, L), jnp.int32),
            pltpu.VMEM_SHARED((PAD,), jnp.int32),
        ],
    )
    return f(pe_flat)


def _gather_body(PAD, src_hbm, x_hbm, xs_hbm, src_v, xrow0, xrow1, xrow2, xrow3, sem):
    cid = lax.axis_index("c")
    sid = lax.axis_index("s")
    wid = sid * NC + cid
    spw = PAD // NW
    base = wid * spw
    pltpu.sync_copy(src_hbm.at[pl.ds(base, spw)], src_v)
    bufs = (xrow0, xrow1, xrow2, xrow3)
    nb = len(bufs)

    @pl.loop(0, spw // (L * nb))
    def _(g):
        j0 = g * nb
        started = []
        for b in range(nb):
            idxv = src_v[pl.ds((j0 + b) * L, L)]
            started.append(pltpu.async_copy(x_hbm.at[idxv], bufs[b], sem))
        for b in range(nb):
            started[b].wait()
            pltpu.sync_copy(bufs[b], xs_hbm.at[pl.ds(base + (j0 + b) * L, L)])


def _gather(src, x_flat):
    PAD = src.shape[0]
    mesh = plsc.VectorSubcoreMesh(core_axis_name="c", subcore_axis_name="s", num_cores=NC, num_subcores=NS)
    f = pl.kernel(
        functools.partial(_gather_body, PAD),
        compiler_params=pltpu.CompilerParams(needs_layout_passes=False),
        out_type=jax.ShapeDtypeStruct((PAD, H), jnp.float32),
        mesh=mesh,
        scratch_types=[
            pltpu.VMEM((PAD // NW,), jnp.int32),
            pltpu.VMEM((L, H), jnp.float32),
            pltpu.VMEM((L, H), jnp.float32),
            pltpu.VMEM((L, H), jnp.float32),
            pltpu.VMEM((L, H), jnp.float32),
            pltpu.SemaphoreType.DMA,
        ],
    )
    return f(src, x_flat)


def _gemm_body(te_ref, xs_ref, wg_ref, wu_ref, wd_ref, y_ref):
    xw = xs_ref[...]
    hg = jnp.dot(xw, wg_ref[0], preferred_element_type=jnp.float32)
    hu = jnp.dot(xw, wu_ref[0], preferred_element_type=jnp.float32)
    h1 = hg * jax.nn.sigmoid(hg) * hu
    y_ref[...] = jnp.dot(h1, wd_ref[0], preferred_element_type=jnp.float32)


def _gemm(te, xs, W_gate, W_up, W_down):
    PAD = xs.shape[0]
    ntiles = PAD // TT
    grid_spec = pltpu.PrefetchScalarGridSpec(
        num_scalar_prefetch=1,
        grid=(ntiles,),
        in_specs=[
            pl.BlockSpec((TT, H), lambda i, te_ref: (i, 0)),
            pl.BlockSpec((1, H, FFE), lambda i, te_ref: (te_ref[i], 0, 0)),
            pl.BlockSpec((1, H, FFE), lambda i, te_ref: (te_ref[i], 0, 0)),
            pl.BlockSpec((1, FFE, H), lambda i, te_ref: (te_ref[i], 0, 0)),
        ],
        out_specs=pl.BlockSpec((TT, H), lambda i, te_ref: (i, 0)),
    )
    return pl.pallas_call(
        _gemm_body,
        grid_spec=grid_spec,
        out_shape=jax.ShapeDtypeStruct((PAD, H), jnp.float32),
    )(te, xs, W_gate, W_up, W_down)


def _combine_body(T, pos_hbm, pw_hbm, y_hbm, out_hbm,
                  pos1_v, pos2_v, pw1_v, pw2_v, y1_v, y2_v, out_v, sem1, sem2):
    cid = lax.axis_index("c")
    sid = lax.axis_index("s")
    wid = sid * NC + cid
    tpw = T // NW
    tb = wid * tpw
    pltpu.sync_copy(pos_hbm.at[pl.ds(tb, tpw)], pos1_v)
    pltpu.sync_copy(pos_hbm.at[pl.ds(T + tb, tpw)], pos2_v)
    pltpu.sync_copy(pw_hbm.at[pl.ds(tb, tpw)], pw1_v.at[pl.ds(0, tpw)])
    pltpu.sync_copy(pw_hbm.at[pl.ds(T + tb, tpw)], pw2_v.at[pl.ds(0, tpw)])

    @pl.loop(0, tpw // L)
    def _(jc):
        i1v = pos1_v[pl.ds(jc * L, L)]
        i2v = pos2_v[pl.ds(jc * L, L)]
        d1 = pltpu.async_copy(y_hbm.at[i1v], y1_v, sem1)
        d2 = pltpu.async_copy(y_hbm.at[i2v], y2_v, sem2)
        d1.wait()
        d2.wait()
        for r in range(L):
            w1 = pw1_v[pl.ds(jc * L + r, L)][0]
            w2 = pw2_v[pl.ds(jc * L + r, L)][0]

            @pl.loop(0, H // L)
            def _(c):
                out_v[r, pl.ds(c * L, L)] = (
                    y1_v[r, pl.ds(c * L, L)] * w1 + y2_v[r, pl.ds(c * L, L)] * w2
                )
        pltpu.sync_copy(out_v, out_hbm.at[pl.ds(tb + jc * L, L)])


def _combine(pos, pw_flat, y, T):
    mesh = plsc.VectorSubcoreMesh(core_axis_name="c", subcore_axis_name="s", num_cores=NC, num_subcores=NS)
    f = pl.kernel(
        functools.partial(_combine_body, T),
        compiler_params=pltpu.CompilerParams(needs_layout_passes=False),
        out_type=jax.ShapeDtypeStruct((T, H), jnp.float32),
        mesh=mesh,
        scratch_types=[
            pltpu.VMEM((T // NW,), jnp.int32),
            pltpu.VMEM((T // NW,), jnp.int32),
            pltpu.VMEM((T // NW + L,), jnp.float32),
            pltpu.VMEM((T // NW + L,), jnp.float32),
            pltpu.VMEM((L, H), jnp.float32),
            pltpu.VMEM((L, H), jnp.float32),
            pltpu.VMEM((L, H), jnp.float32),
            pltpu.SemaphoreType.DMA,
            pltpu.SemaphoreType.DMA,
        ],
    )
    return f(pos, pw_flat, y)


@jax.jit
def kernel(x, gate_weight, W_gate, W_up, W_down):
    b, s, h = x.shape
    T = b * s
    x_flat = x.reshape(T, h)
    pe1, pe2, pw1, pw2 = _router(x_flat, gate_weight)
    pe_flat = jnp.concatenate([pe1.reshape(-1), pe2.reshape(-1)])
    pw_flat = jnp.concatenate([pw1.reshape(-1), pw2.reshape(-1)])
    pos, src, te = _dispatch(pe_flat)
    xs = _gather(src, x_flat)
    y = _gemm(te, xs, W_gate, W_up, W_down)
    out = _combine(pos, pw_flat, y, T)
    return out.reshape(b, s, h)
